# baseline (speedup 1.0000x reference)
import functools

import jax
import jax.numpy as jnp
from jax import lax
from jax.experimental import pallas as pl
from jax.experimental.pallas import tpu as pltpu
from jax.experimental.pallas import tpu_sc as plsc

N = 10000
E = 320000
D = 128
D_EDGE = 4
HID = 128

NSEG = 4
ES = E // NSEG

NC = 2
NS = 16
NW = NC * NS
CHK = 128
QK = 32
NCHT = ES // CHK
NFULL = NCHT // NW
NTAIL = NCHT - NFULL * NW
L = 16

NACC = N
RPT = 640
RPT15 = NACC - 15 * RPT

_mesh = plsc.VectorSubcoreMesh(core_axis_name="c", subcore_axis_name="s")


@functools.partial(
    pl.kernel,
    mesh=_mesh,
    compiler_params=pltpu.CompilerParams(needs_layout_passes=False),
    out_type=[
        jax.ShapeDtypeStruct((ES, D), jnp.float32),
        jax.ShapeDtypeStruct((ES, D), jnp.float32),
        jax.ShapeDtypeStruct((4, ES), jnp.float32),
    ],
    scratch_types=[
        [pltpu.VMEM((CHK,), jnp.int32) for _ in range(4)],
        [pltpu.VMEM((CHK, D), jnp.float32) for _ in range(4)],
        [pltpu.VMEM((4, CHK), jnp.float32) for _ in range(2)],
        pltpu.VMEM((3 * N,), jnp.float32),
        [pltpu.SemaphoreType.DMA for _ in range(4)],
    ],
)
def _gather(h_hbm, coordf_hbm, row_hbm, col_hbm, outr, outc, outd,
            idx, rows, dr, coord_v, sems):
    wid = lax.axis_index("s") * NC + lax.axis_index("c")
    pltpu.sync_copy(coordf_hbm, coord_v)
    tw = NFULL + jnp.where(wid < NTAIL, 1, 0)

    def slot_ck(k):
        return jnp.where(k < NFULL, k * NW + wid, NFULL * NW + wid)

    def regwork(idx_r, idx_c, dr_v, base):
        for g in range(CHK // L):
            ri3 = idx_r[pl.ds(g * L, L)] * 3
            ci3 = idx_c[pl.ds(g * L, L)] * 3
            dx = (plsc.load_gather(coord_v, [ri3])
                  - plsc.load_gather(coord_v, [ci3]))
            dy = (plsc.load_gather(coord_v, [ri3 + 1])
                  - plsc.load_gather(coord_v, [ci3 + 1]))
            dz = (plsc.load_gather(coord_v, [ri3 + 2])
                  - plsc.load_gather(coord_v, [ci3 + 2]))
            dr_v[0, pl.ds(g * L, L)] = dx
            dr_v[1, pl.ds(g * L, L)] = dy
            dr_v[2, pl.ds(g * L, L)] = dz
            dr_v[3, pl.ds(g * L, L)] = dx * dx + dy * dy + dz * dz
        pltpu.sync_copy(dr_v, outd.at[:, pl.ds(base, CHK)])

    def pair(m, carry):
        k0 = 2 * m
        k1 = 2 * m + 1
        b0 = slot_ck(k0) * CHK
        b1 = slot_ck(k1) * CHK
        has1 = k1 < tw

        a0 = pltpu.async_copy(row_hbm.at[pl.ds(b0, CHK)], idx[0], sems[0])
        a1 = pltpu.async_copy(col_hbm.at[pl.ds(b0, CHK)], idx[1], sems[1])
        a0.wait()
        g0 = pltpu.async_copy(h_hbm.at[idx[0]], rows[0], sems[0])
        a1.wait()
        g1 = pltpu.async_copy(h_hbm.at[idx[1]], rows[1], sems[1])

        @pl.when(has1)
        def _():
            a2 = pltpu.async_copy(row_hbm.at[pl.ds(b1, CHK)], idx[2],
                                  sems[2])
            a3 = pltpu.async_copy(col_hbm.at[pl.ds(b1, CHK)], idx[3],
                                  sems[3])
            a2.wait()
            pltpu.async_copy(h_hbm.at[idx[2]], rows[2], sems[2])
            a3.wait()
            pltpu.async_copy(h_hbm.at[idx[3]], rows[3], sems[3])

        regwork(idx[0], idx[1], dr[0], b0)
        g0.wait()
        o0 = pltpu.async_copy(rows[0], outr.at[pl.ds(b0, CHK)], sems[0])
        g1.wait()
        o1 = pltpu.async_copy(rows[1], outc.at[pl.ds(b0, CHK)], sems[1])

        @pl.when(has1)
        def _():
            regwork(idx[2], idx[3], dr[1], b1)
            pltpu.make_async_copy(h_hbm.at[idx[2]], rows[2], sems[2]).wait()
            pltpu.async_copy(rows[2], outr.at[pl.ds(b1, CHK)], sems[2])
            pltpu.make_async_copy(h_hbm.at[idx[3]], rows[3], sems[3]).wait()
            pltpu.async_copy(rows[3], outc.at[pl.ds(b1, CHK)], sems[3])
            pltpu.make_async_copy(rows[2], outr.at[pl.ds(b1, CHK)],
                                  sems[2]).wait()
            pltpu.make_async_copy(rows[3], outc.at[pl.ds(b1, CHK)],
                                  sems[3]).wait()

        o0.wait()
        o1.wait()
        return carry

    lax.fori_loop(0, (NFULL + 2) // 2, pair, 0)


@functools.partial(
    pl.kernel,
    mesh=_mesh,
    compiler_params=pltpu.CompilerParams(needs_layout_passes=False),
    out_type=[
        jax.ShapeDtypeStruct((NC, NACC, D), jnp.float32),
        jax.ShapeDtypeStruct((NW, 4 * N), jnp.float32),
    ],
    scratch_types=[
        pltpu.VMEM((CHK // QK, QK), jnp.int32),
        pltpu.VMEM((QK, D), jnp.float32),
        pltpu.VMEM((QK, D), jnp.float32),
        pltpu.VMEM((4, CHK), jnp.float32),
        pltpu.VMEM((4 * N,), jnp.float32),
        pltpu.VMEM_SHARED((NACC, D), jnp.float32),
        pltpu.SemaphoreType.DMA,
        pltpu.SemaphoreType.DMA,
        pltpu.SemaphoreType.DMA,
        pltpu.SemaphoreType.DMA,
    ],
)
def _scatter(ef_hbm, tdp_hbm, row2_hbm, z2_hbm, zf_hbm, outh, out4,
             idx2, data0, data1, td_v, acc4_v, acch_sh,
             sem_i, sem_t, sem_d0, sem_d1):
    cid = lax.axis_index("c")
    sid = lax.axis_index("s")
    wid = sid * NC + cid

    @pl.when(sid < NS - 1)
    def _():
        pltpu.sync_copy(z2_hbm, acch_sh.at[pl.ds(sid * RPT, RPT)])

    @pl.when(sid == NS - 1)
    def _():
        pltpu.sync_copy(z2_hbm.at[pl.ds(0, RPT15)],
                        acch_sh.at[pl.ds((NS - 1) * RPT, RPT15)])

    pltpu.sync_copy(zf_hbm, acc4_v)
    plsc.subcore_barrier()

    def reg_scatter(klo, khi):
        for k in range(klo, khi):
            ids = idx2[k // 2, pl.ds((k % 2) * L, L)]
            for p in range(4):
                val = td_v[p, pl.ds(k * L, L)]
                plsc.addupdate_scatter(acc4_v, [ids + p * N], val)

    def chunk(ck):
        base = ck * CHK
        rbase = ck * (CHK // QK)
        a_i = pltpu.async_copy(row2_hbm.at[pl.ds(rbase, CHK // QK)], idx2,
                               sem_i)
        a_t = pltpu.async_copy(tdp_hbm.at[:, pl.ds(base, CHK)], td_v, sem_t)
        bufs = (data0, data1)
        sems = (sem_d0, sem_d1)
        pend = [
            pltpu.async_copy(ef_hbm.at[pl.ds(base, QK)], data0, sem_d0),
            pltpu.async_copy(ef_hbm.at[pl.ds(base + QK, QK)], data1, sem_d1),
        ]
        a_i.wait()
        a_t.wait()
        nq = CHK // QK
        for q in range(nq):
            pend[q].wait()
            pltpu.sync_copy(bufs[q % 2], acch_sh.at[idx2.at[q]], add=True)
            if q + 2 < nq:
                pend.append(pltpu.async_copy(
                    ef_hbm.at[pl.ds(base + (q + 2) * QK, QK)],
                    bufs[q % 2], sems[q % 2]))
            reg_scatter(2 * q, 2 * q + 2)

    def body(j, carry):
        chunk(j * NW + wid)
        return carry

    lax.fori_loop(0, NFULL, body, 0)

    @pl.when(wid < NTAIL)
    def _():
        chunk(NFULL * NW + wid)

    pltpu.sync_copy(acc4_v, out4.at[wid])
    plsc.subcore_barrier()

    @pl.when(sid < NS - 1)
    def _():
        off = sid * RPT
        pltpu.sync_copy(acch_sh.at[pl.ds(off, RPT)],
                        outh.at[cid, pl.ds(off, RPT)])

    @pl.when(sid == NS - 1)
    def _():
        off = (NS - 1) * RPT
        pltpu.sync_copy(acch_sh.at[pl.ds(off, RPT15)],
                        outh.at[cid, pl.ds(off, RPT15)])


def _silu(x):
    return x * (1.0 / (1.0 + jnp.exp(-x)))


def _bdot(a, b):
    return jnp.dot(a.astype(jnp.bfloat16), b,
                   preferred_element_type=jnp.float32)


T_E = 3200
G_E = ES // T_E


def _edge_body(hr, hc, drp, eap, w1h, w1c, w4, w1e, be1, we2, be2, wc1, bc1,
               wc2, ef_out, tdp_out):
    drb = drp[...]
    x = (_bdot(hr[...], w1h[...])
         + _bdot(hc[...], w1c[...])
         + lax.dot_general(drb, w4[...], (((0,), (0,)), ((), ())),
                           preferred_element_type=jnp.float32)
         + lax.dot_general(eap[...], w1e[...], (((0,), (0,)), ((), ())),
                           preferred_element_type=jnp.float32)
         + be1[...])
    x = _silu(x)
    ef = _silu(_bdot(x, we2[...]) + be2[...])
    cf = _silu(_bdot(ef, wc1[...]) + bc1[...])
    s = lax.dot_general(wc2[...], cf, (((1,), (1,)), ((), ())),
                        preferred_element_type=jnp.float32)
    tr = jnp.clip(drb[:3, :] * s, -100.0, 100.0)
    ef_out[...] = ef
    tdp_out[:3, :] = tr
    tdp_out[3:4, :] = jnp.ones((1, T_E), jnp.float32)


def _edge_mlp(hr, hc, drp, eap, w1h, w1c, w4, w1e, be1, we2, be2, wc1, bc1,
              wc2):
    full = lambda shape: pl.BlockSpec(shape, lambda i: (0, 0))
    return pl.pallas_call(
        _edge_body,
        grid=(G_E,),
        in_specs=[
            pl.BlockSpec((T_E, D), lambda i: (i, 0)),
            pl.BlockSpec((T_E, D), lambda i: (i, 0)),
            pl.BlockSpec((4, T_E), lambda i: (0, i)),
            pl.BlockSpec((D_EDGE, T_E), lambda i: (0, i)),
            full((D, HID)),
            full((D, HID)),
            full((4, HID)),
            full((D_EDGE, HID)),
            full((1, HID)),
            full((HID, HID)),
            full((1, HID)),
            full((HID, HID)),
            full((1, HID)),
            full((1, HID)),
        ],
        out_specs=[
            pl.BlockSpec((T_E, D), lambda i: (i, 0)),
            pl.BlockSpec((4, T_E), lambda i: (0, i)),
        ],
        out_shape=[
            jax.ShapeDtypeStruct((ES, D), jnp.float32),
            jax.ShapeDtypeStruct((4, ES), jnp.float32),
        ],
        compiler_params=pltpu.CompilerParams(
            dimension_semantics=("arbitrary",)),
    )(hr, hc, drp, eap, w1h, w1c, w4, w1e, be1, we2, be2, wc1, bc1, wc2)


T_N = 2000
G_N = N // T_N


def _node_body(h, coordp, *rest):
    acchs = rest[:NSEG]
    acc4s = rest[NSEG:2 * NSEG]
    wn1h, wn1a, bn1, wn2, bn2, h_out, coordp_out = rest[2 * NSEG:]
    aggh = acchs[0][0] + acchs[0][1]
    for s in range(1, NSEG):
        aggh = aggh + acchs[s][0] + acchs[s][1]
    y = _silu(_bdot(h[...], wn1h[...]) + _bdot(aggh, wn1a[...]) + bn1[...])
    h_out[...] = h[...] + _bdot(y, wn2[...]) + bn2[...]

    @pl.when(pl.program_id(0) == 0)
    def _():
        a4 = acc4s[0][0]
        for k in range(1, NW):
            a4 = a4 + acc4s[0][k]
        for s in range(1, NSEG):
            for k in range(NW):
                a4 = a4 + acc4s[s][k]
        num = a4[:3, :]
        cnt = a4[3:4, :]
        coordp_out[...] = coordp[...] + num / jnp.maximum(cnt, 1.0)


def _node_mlp(h, coordp, acchs, acc4s, wn1h, wn1a, bn1, wn2, bn2):
    full = lambda shape: pl.BlockSpec(shape, lambda i: (0, 0))
    acch_spec = pl.BlockSpec((NC, T_N, D), lambda i: (0, i, 0))
    acc4_spec = pl.BlockSpec((NW, 4, N), lambda i: (0, 0, 0))
    return pl.pallas_call(
        _node_body,
        grid=(G_N,),
        in_specs=[
            pl.BlockSpec((T_N, D), lambda i: (i, 0)),
            pl.BlockSpec((3, N), lambda i: (0, 0)),
        ] + [acch_spec] * NSEG + [acc4_spec] * NSEG + [
            full((D, HID)),
            full((D, HID)),
            full((1, HID)),
            full((HID, D)),
            full((1, D)),
        ],
        out_specs=[
            pl.BlockSpec((T_N, D), lambda i: (i, 0)),
            pl.BlockSpec((3, N), lambda i: (0, 0)),
        ],
        out_shape=[
            jax.ShapeDtypeStruct((N, D), jnp.float32),
            jax.ShapeDtypeStruct((3, N), jnp.float32),
        ],
        compiler_params=pltpu.CompilerParams(
            dimension_semantics=("arbitrary",)),
    )(h, coordp, *acchs, *acc4s, wn1h, wn1a, bn1, wn2, bn2)


def kernel(h, coord, edge_attr, W_e1, b_e1, W_e2, b_e2, W_n1, b_n1, W_n2,
           b_n2, W_c1, b_c1, W_c2, edge_index):
    bf = jnp.bfloat16
    coordf = coord.reshape(-1)
    eap = edge_attr.T

    w4 = jnp.concatenate(
        [jnp.zeros((3, HID), jnp.float32), W_e1[2 * D:2 * D + 1]], axis=0)
    ew = (W_e1[:D].astype(bf), W_e1[D:2 * D].astype(bf), w4,
          W_e1[2 * D + 1:], b_e1.reshape(1, HID),
          W_e2.astype(bf), b_e2.reshape(1, HID),
          W_c1.astype(bf), b_c1.reshape(1, HID), W_c2.reshape(1, HID))

    z2 = jnp.zeros((RPT, D), jnp.float32)
    zf = jnp.zeros((4 * N,), jnp.float32)

    rows = [edge_index[0, s * ES:(s + 1) * ES] for s in range(NSEG)]
    cols = [edge_index[1, s * ES:(s + 1) * ES] for s in range(NSEG)]
    eaps = [eap[:, s * ES:(s + 1) * ES] for s in range(NSEG)]

    gat = [_gather(h, coordf, rows[s], cols[s]) for s in range(NSEG)]
    edg = [_edge_mlp(gat[s][0], gat[s][1], gat[s][2], eaps[s], *ew)
           for s in range(NSEG)]
    sca = [_scatter(edg[s][0], edg[s][1], rows[s].reshape(ES // QK, QK),
                    z2, zf) for s in range(NSEG)]

    h_out, coordp_out = _node_mlp(
        h, coord.T, [s[0] for s in sca],
        [s[1].reshape(NW, 4, N) for s in sca],
        W_n1[:D].astype(bf), W_n1[D:].astype(bf), b_n1.reshape(1, HID),
        W_n2.astype(bf), b_n2.reshape(1, D))
    return (h_out, coordp_out.T)


# speedup vs baseline: 1.1581x; 1.1581x over previous
import functools

import jax
import jax.numpy as jnp
from jax import lax
from jax.experimental import pallas as pl
from jax.experimental.pallas import tpu as pltpu
from jax.experimental.pallas import tpu_sc as plsc

N = 10000
E = 320000
D = 128
D_EDGE = 4
HID = 128

NSEG = 2
ES = E // NSEG

NC = 2
NS = 16
NW = NC * NS
CHK = 128
QK = 32
NCHT = ES // CHK
NFULL = NCHT // NW
NTAIL = NCHT - NFULL * NW
L = 16

NACC = N
RPT = 640
RPT15 = NACC - 15 * RPT

_mesh = plsc.VectorSubcoreMesh(core_axis_name="c", subcore_axis_name="s")


@functools.partial(
    pl.kernel,
    mesh=_mesh,
    compiler_params=pltpu.CompilerParams(needs_layout_passes=False),
    out_type=[
        jax.ShapeDtypeStruct((ES, D), jnp.float32),
        jax.ShapeDtypeStruct((ES, D), jnp.float32),
        jax.ShapeDtypeStruct((4, ES), jnp.float32),
    ],
    scratch_types=[
        [pltpu.VMEM((CHK,), jnp.int32) for _ in range(4)],
        [pltpu.VMEM((CHK, D), jnp.float32) for _ in range(4)],
        [pltpu.VMEM((4, CHK), jnp.float32) for _ in range(2)],
        pltpu.VMEM((3 * N,), jnp.float32),
        [pltpu.SemaphoreType.DMA for _ in range(4)],
    ],
)
def _gather(h_hbm, coordf_hbm, row_hbm, col_hbm, outr, outc, outd,
            idx, rows, dr, coord_v, sems):
    wid = lax.axis_index("s") * NC + lax.axis_index("c")
    pltpu.sync_copy(coordf_hbm, coord_v)
    tw = NFULL + jnp.where(wid < NTAIL, 1, 0)

    def slot_ck(k):
        return jnp.where(k < NFULL, k * NW + wid, NFULL * NW + wid)

    def regwork(idx_r, idx_c, dr_v, base):
        for g in range(CHK // L):
            ri3 = idx_r[pl.ds(g * L, L)] * 3
            ci3 = idx_c[pl.ds(g * L, L)] * 3
            dx = (plsc.load_gather(coord_v, [ri3])
                  - plsc.load_gather(coord_v, [ci3]))
            dy = (plsc.load_gather(coord_v, [ri3 + 1])
                  - plsc.load_gather(coord_v, [ci3 + 1]))
            dz = (plsc.load_gather(coord_v, [ri3 + 2])
                  - plsc.load_gather(coord_v, [ci3 + 2]))
            dr_v[0, pl.ds(g * L, L)] = dx
            dr_v[1, pl.ds(g * L, L)] = dy
            dr_v[2, pl.ds(g * L, L)] = dz
            dr_v[3, pl.ds(g * L, L)] = dx * dx + dy * dy + dz * dz
        pltpu.sync_copy(dr_v, outd.at[:, pl.ds(base, CHK)])

    def pair(m, carry):
        k0 = 2 * m
        k1 = 2 * m + 1
        b0 = slot_ck(k0) * CHK
        b1 = slot_ck(k1) * CHK
        has1 = k1 < tw

        a0 = pltpu.async_copy(row_hbm.at[pl.ds(b0, CHK)], idx[0], sems[0])
        a1 = pltpu.async_copy(col_hbm.at[pl.ds(b0, CHK)], idx[1], sems[1])
        a0.wait()
        g0 = pltpu.async_copy(h_hbm.at[idx[0]], rows[0], sems[0])
        a1.wait()
        g1 = pltpu.async_copy(h_hbm.at[idx[1]], rows[1], sems[1])

        @pl.when(has1)
        def _():
            a2 = pltpu.async_copy(row_hbm.at[pl.ds(b1, CHK)], idx[2],
                                  sems[2])
            a3 = pltpu.async_copy(col_hbm.at[pl.ds(b1, CHK)], idx[3],
                                  sems[3])
            a2.wait()
            pltpu.async_copy(h_hbm.at[idx[2]], rows[2], sems[2])
            a3.wait()
            pltpu.async_copy(h_hbm.at[idx[3]], rows[3], sems[3])

        regwork(idx[0], idx[1], dr[0], b0)
        g0.wait()
        o0 = pltpu.async_copy(rows[0], outr.at[pl.ds(b0, CHK)], sems[0])
        g1.wait()
        o1 = pltpu.async_copy(rows[1], outc.at[pl.ds(b0, CHK)], sems[1])

        @pl.when(has1)
        def _():
            regwork(idx[2], idx[3], dr[1], b1)
            pltpu.make_async_copy(h_hbm.at[idx[2]], rows[2], sems[2]).wait()
            pltpu.async_copy(rows[2], outr.at[pl.ds(b1, CHK)], sems[2])
            pltpu.make_async_copy(h_hbm.at[idx[3]], rows[3], sems[3]).wait()
            pltpu.async_copy(rows[3], outc.at[pl.ds(b1, CHK)], sems[3])
            pltpu.make_async_copy(rows[2], outr.at[pl.ds(b1, CHK)],
                                  sems[2]).wait()
            pltpu.make_async_copy(rows[3], outc.at[pl.ds(b1, CHK)],
                                  sems[3]).wait()

        o0.wait()
        o1.wait()
        return carry

    lax.fori_loop(0, (NFULL + 2) // 2, pair, 0)


@functools.partial(
    pl.kernel,
    mesh=_mesh,
    compiler_params=pltpu.CompilerParams(needs_layout_passes=False),
    out_type=[
        jax.ShapeDtypeStruct((NC, NACC, D), jnp.float32),
        jax.ShapeDtypeStruct((NW, 4 * N), jnp.float32),
    ],
    scratch_types=[
        pltpu.VMEM((CHK // QK, QK), jnp.int32),
        pltpu.VMEM((QK, D), jnp.float32),
        pltpu.VMEM((QK, D), jnp.float32),
        pltpu.VMEM((4, CHK), jnp.float32),
        pltpu.VMEM((4 * N,), jnp.float32),
        pltpu.VMEM_SHARED((NACC, D), jnp.float32),
        pltpu.SemaphoreType.DMA,
        pltpu.SemaphoreType.DMA,
        pltpu.SemaphoreType.DMA,
        pltpu.SemaphoreType.DMA,
    ],
)
def _scatter(ef_hbm, tdp_hbm, row2_hbm, z2_hbm, zf_hbm, outh, out4,
             idx2, data0, data1, td_v, acc4_v, acch_sh,
             sem_i, sem_t, sem_d0, sem_d1):
    cid = lax.axis_index("c")
    sid = lax.axis_index("s")
    wid = sid * NC + cid

    @pl.when(sid < NS - 1)
    def _():
        pltpu.sync_copy(z2_hbm, acch_sh.at[pl.ds(sid * RPT, RPT)])

    @pl.when(sid == NS - 1)
    def _():
        pltpu.sync_copy(z2_hbm.at[pl.ds(0, RPT15)],
                        acch_sh.at[pl.ds((NS - 1) * RPT, RPT15)])

    pltpu.sync_copy(zf_hbm, acc4_v)
    plsc.subcore_barrier()

    def reg_scatter(klo, khi):
        for k in range(klo, khi):
            ids = idx2[k // 2, pl.ds((k % 2) * L, L)]
            for p in range(4):
                val = td_v[p, pl.ds(k * L, L)]
                plsc.addupdate_scatter(acc4_v, [ids + p * N], val)

    def chunk(ck):
        base = ck * CHK
        rbase = ck * (CHK // QK)
        a_i = pltpu.async_copy(row2_hbm.at[pl.ds(rbase, CHK // QK)], idx2,
                               sem_i)
        a_t = pltpu.async_copy(tdp_hbm.at[:, pl.ds(base, CHK)], td_v, sem_t)
        bufs = (data0, data1)
        sems = (sem_d0, sem_d1)
        pend = [
            pltpu.async_copy(ef_hbm.at[pl.ds(base, QK)], data0, sem_d0),
            pltpu.async_copy(ef_hbm.at[pl.ds(base + QK, QK)], data1, sem_d1),
        ]
        a_i.wait()
        a_t.wait()
        nq = CHK // QK
        for q in range(nq):
            pend[q].wait()
            pltpu.sync_copy(bufs[q % 2], acch_sh.at[idx2.at[q]], add=True)
            if q + 2 < nq:
                pend.append(pltpu.async_copy(
                    ef_hbm.at[pl.ds(base + (q + 2) * QK, QK)],
                    bufs[q % 2], sems[q % 2]))
            reg_scatter(2 * q, 2 * q + 2)

    def body(j, carry):
        chunk(j * NW + wid)
        return carry

    lax.fori_loop(0, NFULL, body, 0)

    @pl.when(wid < NTAIL)
    def _():
        chunk(NFULL * NW + wid)

    pltpu.sync_copy(acc4_v, out4.at[wid])
    plsc.subcore_barrier()

    @pl.when(sid < NS - 1)
    def _():
        off = sid * RPT
        pltpu.sync_copy(acch_sh.at[pl.ds(off, RPT)],
                        outh.at[cid, pl.ds(off, RPT)])

    @pl.when(sid == NS - 1)
    def _():
        off = (NS - 1) * RPT
        pltpu.sync_copy(acch_sh.at[pl.ds(off, RPT15)],
                        outh.at[cid, pl.ds(off, RPT15)])


def _silu(x):
    return x * (1.0 / (1.0 + jnp.exp(-x)))


def _bdot(a, b):
    return jnp.dot(a.astype(jnp.bfloat16), b,
                   preferred_element_type=jnp.float32)


T_E = 3200
G_E = ES // T_E


def _edge_body(hr, hc, drp, eap, w1h, w1c, w4, w1e, be1, we2, be2, wc1, bc1,
               wc2, ef_out, tdp_out):
    drb = drp[...]
    x = (_bdot(hr[...], w1h[...])
         + _bdot(hc[...], w1c[...])
         + lax.dot_general(drb, w4[...], (((0,), (0,)), ((), ())),
                           preferred_element_type=jnp.float32)
         + lax.dot_general(eap[...], w1e[...], (((0,), (0,)), ((), ())),
                           preferred_element_type=jnp.float32)
         + be1[...])
    x = _silu(x)
    ef = _silu(_bdot(x, we2[...]) + be2[...])
    cf = _silu(_bdot(ef, wc1[...]) + bc1[...])
    s = lax.dot_general(wc2[...], cf, (((1,), (1,)), ((), ())),
                        preferred_element_type=jnp.float32)
    tr = jnp.clip(drb[:3, :] * s, -100.0, 100.0)
    ef_out[...] = ef
    tdp_out[:3, :] = tr
    tdp_out[3:4, :] = jnp.ones((1, T_E), jnp.float32)


def _edge_mlp(hr, hc, drp, eap, w1h, w1c, w4, w1e, be1, we2, be2, wc1, bc1,
              wc2):
    full = lambda shape: pl.BlockSpec(shape, lambda i: (0, 0))
    return pl.pallas_call(
        _edge_body,
        grid=(G_E,),
        in_specs=[
            pl.BlockSpec((T_E, D), lambda i: (i, 0)),
            pl.BlockSpec((T_E, D), lambda i: (i, 0)),
            pl.BlockSpec((4, T_E), lambda i: (0, i)),
            pl.BlockSpec((D_EDGE, T_E), lambda i: (0, i)),
            full((D, HID)),
            full((D, HID)),
            full((4, HID)),
            full((D_EDGE, HID)),
            full((1, HID)),
            full((HID, HID)),
            full((1, HID)),
            full((HID, HID)),
            full((1, HID)),
            full((1, HID)),
        ],
        out_specs=[
            pl.BlockSpec((T_E, D), lambda i: (i, 0)),
            pl.BlockSpec((4, T_E), lambda i: (0, i)),
        ],
        out_shape=[
            jax.ShapeDtypeStruct((ES, D), jnp.float32),
            jax.ShapeDtypeStruct((4, ES), jnp.float32),
        ],
        compiler_params=pltpu.CompilerParams(
            dimension_semantics=("arbitrary",)),
    )(hr, hc, drp, eap, w1h, w1c, w4, w1e, be1, we2, be2, wc1, bc1, wc2)


T_N = 2000
G_N = N // T_N


def _node_body(h, coordp, *rest):
    acchs = rest[:NSEG]
    acc4s = rest[NSEG:2 * NSEG]
    wn1h, wn1a, bn1, wn2, bn2, h_out, coordp_out = rest[2 * NSEG:]
    aggh = acchs[0][0] + acchs[0][1]
    for s in range(1, NSEG):
        aggh = aggh + acchs[s][0] + acchs[s][1]
    y = _silu(_bdot(h[...], wn1h[...]) + _bdot(aggh, wn1a[...]) + bn1[...])
    h_out[...] = h[...] + _bdot(y, wn2[...]) + bn2[...]

    @pl.when(pl.program_id(0) == 0)
    def _():
        a4 = acc4s[0][0]
        for k in range(1, NW):
            a4 = a4 + acc4s[0][k]
        for s in range(1, NSEG):
            for k in range(NW):
                a4 = a4 + acc4s[s][k]
        num = a4[:3, :]
        cnt = a4[3:4, :]
        coordp_out[...] = coordp[...] + num / jnp.maximum(cnt, 1.0)


def _node_mlp(h, coordp, acchs, acc4s, wn1h, wn1a, bn1, wn2, bn2):
    full = lambda shape: pl.BlockSpec(shape, lambda i: (0, 0))
    acch_spec = pl.BlockSpec((NC, T_N, D), lambda i: (0, i, 0))
    acc4_spec = pl.BlockSpec((NW, 4, N), lambda i: (0, 0, 0))
    return pl.pallas_call(
        _node_body,
        grid=(G_N,),
        in_specs=[
            pl.BlockSpec((T_N, D), lambda i: (i, 0)),
            pl.BlockSpec((3, N), lambda i: (0, 0)),
        ] + [acch_spec] * NSEG + [acc4_spec] * NSEG + [
            full((D, HID)),
            full((D, HID)),
            full((1, HID)),
            full((HID, D)),
            full((1, D)),
        ],
        out_specs=[
            pl.BlockSpec((T_N, D), lambda i: (i, 0)),
            pl.BlockSpec((3, N), lambda i: (0, 0)),
        ],
        out_shape=[
            jax.ShapeDtypeStruct((N, D), jnp.float32),
            jax.ShapeDtypeStruct((3, N), jnp.float32),
        ],
        compiler_params=pltpu.CompilerParams(
            dimension_semantics=("arbitrary",)),
    )(h, coordp, *acchs, *acc4s, wn1h, wn1a, bn1, wn2, bn2)


def kernel(h, coord, edge_attr, W_e1, b_e1, W_e2, b_e2, W_n1, b_n1, W_n2,
           b_n2, W_c1, b_c1, W_c2, edge_index):
    bf = jnp.bfloat16
    coordf = coord.reshape(-1)
    eap = edge_attr.T

    w4 = jnp.concatenate(
        [jnp.zeros((3, HID), jnp.float32), W_e1[2 * D:2 * D + 1]], axis=0)
    ew = (W_e1[:D].astype(bf), W_e1[D:2 * D].astype(bf), w4,
          W_e1[2 * D + 1:], b_e1.reshape(1, HID),
          W_e2.astype(bf), b_e2.reshape(1, HID),
          W_c1.astype(bf), b_c1.reshape(1, HID), W_c2.reshape(1, HID))

    z2 = jnp.zeros((RPT, D), jnp.float32)
    zf = jnp.zeros((4 * N,), jnp.float32)

    rows = [edge_index[0, s * ES:(s + 1) * ES] for s in range(NSEG)]
    cols = [edge_index[1, s * ES:(s + 1) * ES] for s in range(NSEG)]
    eaps = [eap[:, s * ES:(s + 1) * ES] for s in range(NSEG)]

    gat = [_gather(h, coordf, rows[s], cols[s]) for s in range(NSEG)]
    edg = [_edge_mlp(gat[s][0], gat[s][1], gat[s][2], eaps[s], *ew)
           for s in range(NSEG)]
    sca = [_scatter(edg[s][0], edg[s][1], rows[s].reshape(ES // QK, QK),
                    z2, zf) for s in range(NSEG)]

    h_out, coordp_out = _node_mlp(
        h, coord.T, [s[0] for s in sca],
        [s[1].reshape(NW, 4, N) for s in sca],
        W_n1[:D].astype(bf), W_n1[D:].astype(bf), b_n1.reshape(1, HID),
        W_n2.astype(bf), b_n2.reshape(1, D))
    return (h_out, coordp_out.T)


# speedup vs baseline: 1.1603x; 1.0019x over previous
import functools

import jax
import jax.numpy as jnp
from jax import lax
from jax.experimental import pallas as pl
from jax.experimental.pallas import tpu as pltpu
from jax.experimental.pallas import tpu_sc as plsc

N = 10000
E = 320000
D = 128
D_EDGE = 4
HID = 128

NSEG = 2
ES = E // NSEG

NC = 2
NS = 16
NW = NC * NS
CHK = 128
QK = 32
NCHT = ES // CHK
NFULL = NCHT // NW
NTAIL = NCHT - NFULL * NW
L = 16

NACC = N
RPT = 640
RPT15 = NACC - 15 * RPT

_mesh = plsc.VectorSubcoreMesh(core_axis_name="c", subcore_axis_name="s")


@functools.partial(
    pl.kernel,
    mesh=_mesh,
    compiler_params=pltpu.CompilerParams(needs_layout_passes=False),
    out_type=[
        jax.ShapeDtypeStruct((ES, D), jnp.float32),
        jax.ShapeDtypeStruct((ES, D), jnp.float32),
        jax.ShapeDtypeStruct((4, ES), jnp.float32),
    ],
    scratch_types=[
        [pltpu.VMEM((CHK,), jnp.int32) for _ in range(4)],
        [pltpu.VMEM((CHK, D), jnp.float32) for _ in range(4)],
        [pltpu.VMEM((4, CHK), jnp.float32) for _ in range(2)],
        pltpu.VMEM((3 * N,), jnp.float32),
        [pltpu.SemaphoreType.DMA for _ in range(4)],
    ],
)
def _gather(h_hbm, coordf_hbm, row_hbm, col_hbm, outr, outc, outd,
            idx, rows, dr, coord_v, sems):
    wid = lax.axis_index("s") * NC + lax.axis_index("c")
    pltpu.sync_copy(coordf_hbm, coord_v)
    tw = NFULL + jnp.where(wid < NTAIL, 1, 0)

    def slot_ck(k):
        return jnp.where(k < NFULL, k * NW + wid, NFULL * NW + wid)

    def regwork(idx_r, idx_c, dr_v, base):
        for g in range(CHK // L):
            ri3 = idx_r[pl.ds(g * L, L)] * 3
            ci3 = idx_c[pl.ds(g * L, L)] * 3
            dx = (plsc.load_gather(coord_v, [ri3])
                  - plsc.load_gather(coord_v, [ci3]))
            dy = (plsc.load_gather(coord_v, [ri3 + 1])
                  - plsc.load_gather(coord_v, [ci3 + 1]))
            dz = (plsc.load_gather(coord_v, [ri3 + 2])
                  - plsc.load_gather(coord_v, [ci3 + 2]))
            dr_v[0, pl.ds(g * L, L)] = dx
            dr_v[1, pl.ds(g * L, L)] = dy
            dr_v[2, pl.ds(g * L, L)] = dz
            dr_v[3, pl.ds(g * L, L)] = dx * dx + dy * dy + dz * dz
        pltpu.sync_copy(dr_v, outd.at[:, pl.ds(base, CHK)])

    def pair(m, carry):
        k0 = 2 * m
        k1 = 2 * m + 1
        b0 = slot_ck(k0) * CHK
        b1 = slot_ck(k1) * CHK
        has1 = k1 < tw

        a0 = pltpu.async_copy(row_hbm.at[pl.ds(b0, CHK)], idx[0], sems[0])
        a1 = pltpu.async_copy(col_hbm.at[pl.ds(b0, CHK)], idx[1], sems[1])
        a0.wait()
        g0 = pltpu.async_copy(h_hbm.at[idx[0]], rows[0], sems[0])
        a1.wait()
        g1 = pltpu.async_copy(h_hbm.at[idx[1]], rows[1], sems[1])

        @pl.when(has1)
        def _():
            a2 = pltpu.async_copy(row_hbm.at[pl.ds(b1, CHK)], idx[2],
                                  sems[2])
            a3 = pltpu.async_copy(col_hbm.at[pl.ds(b1, CHK)], idx[3],
                                  sems[3])
            a2.wait()
            pltpu.async_copy(h_hbm.at[idx[2]], rows[2], sems[2])
            a3.wait()
            pltpu.async_copy(h_hbm.at[idx[3]], rows[3], sems[3])

        regwork(idx[0], idx[1], dr[0], b0)
        g0.wait()
        o0 = pltpu.async_copy(rows[0], outr.at[pl.ds(b0, CHK)], sems[0])
        g1.wait()
        o1 = pltpu.async_copy(rows[1], outc.at[pl.ds(b0, CHK)], sems[1])

        @pl.when(has1)
        def _():
            regwork(idx[2], idx[3], dr[1], b1)
            pltpu.make_async_copy(h_hbm.at[idx[2]], rows[2], sems[2]).wait()
            pltpu.async_copy(rows[2], outr.at[pl.ds(b1, CHK)], sems[2])
            pltpu.make_async_copy(h_hbm.at[idx[3]], rows[3], sems[3]).wait()
            pltpu.async_copy(rows[3], outc.at[pl.ds(b1, CHK)], sems[3])
            pltpu.make_async_copy(rows[2], outr.at[pl.ds(b1, CHK)],
                                  sems[2]).wait()
            pltpu.make_async_copy(rows[3], outc.at[pl.ds(b1, CHK)],
                                  sems[3]).wait()

        o0.wait()
        o1.wait()
        return carry

    lax.fori_loop(0, (NFULL + 2) // 2, pair, 0)


@functools.partial(
    pl.kernel,
    mesh=_mesh,
    compiler_params=pltpu.CompilerParams(needs_layout_passes=False),
    out_type=[
        jax.ShapeDtypeStruct((NC, NACC, D), jnp.float32),
        jax.ShapeDtypeStruct((NW, 4 * N), jnp.float32),
    ],
    scratch_types=[
        [pltpu.VMEM((CHK // QK, QK), jnp.int32) for _ in range(2)],
        pltpu.VMEM((QK, D), jnp.float32),
        pltpu.VMEM((QK, D), jnp.float32),
        [pltpu.VMEM((4, CHK), jnp.float32) for _ in range(2)],
        pltpu.VMEM((4 * N,), jnp.float32),
        pltpu.VMEM_SHARED((NACC, D), jnp.float32),
        pltpu.SemaphoreType.DMA,
        pltpu.SemaphoreType.DMA,
        pltpu.SemaphoreType.DMA,
        pltpu.SemaphoreType.DMA,
    ],
)
def _scatter(ef_hbm, tdp_hbm, row2_hbm, z2_hbm, zf_hbm, outh, out4,
             idx2s, data0, data1, td_vs, acc4_v, acch_sh,
             sem_i, sem_t, sem_d0, sem_d1):
    cid = lax.axis_index("c")
    sid = lax.axis_index("s")
    wid = sid * NC + cid

    @pl.when(sid < NS - 1)
    def _():
        pltpu.sync_copy(z2_hbm, acch_sh.at[pl.ds(sid * RPT, RPT)])

    @pl.when(sid == NS - 1)
    def _():
        pltpu.sync_copy(z2_hbm.at[pl.ds(0, RPT15)],
                        acch_sh.at[pl.ds((NS - 1) * RPT, RPT15)])

    pltpu.sync_copy(zf_hbm, acc4_v)
    plsc.subcore_barrier()

    tw = NFULL + jnp.where(wid < NTAIL, 1, 0)

    def slot_ck(k):
        return jnp.where(k < NFULL, k * NW + wid, NFULL * NW + wid)

    def reg_scatter(idx2, td_v, klo, khi):
        for k in range(klo, khi):
            ids = idx2[k // 2, pl.ds((k % 2) * L, L)]
            for p in range(4):
                val = td_v[p, pl.ds(k * L, L)]
                plsc.addupdate_scatter(acc4_v, [ids + p * N], val)

    def issue_hdr(ck, idx2, td_v):
        base = ck * CHK
        rbase = ck * (CHK // QK)
        a_i = pltpu.async_copy(row2_hbm.at[pl.ds(rbase, CHK // QK)], idx2,
                               sem_i)
        a_t = pltpu.async_copy(tdp_hbm.at[:, pl.ds(base, CHK)], td_v, sem_t)
        return a_i, a_t

    def chunk_work(ck, idx2, td_v, a_i, a_t):
        base = ck * CHK
        bufs = (data0, data1)
        sems = (sem_d0, sem_d1)
        pend = [
            pltpu.async_copy(ef_hbm.at[pl.ds(base, QK)], data0, sem_d0),
            pltpu.async_copy(ef_hbm.at[pl.ds(base + QK, QK)], data1, sem_d1),
        ]
        a_i.wait()
        a_t.wait()
        nq = CHK // QK
        for q in range(nq):
            pend[q].wait()
            pltpu.sync_copy(bufs[q % 2], acch_sh.at[idx2.at[q]], add=True)
            if q + 2 < nq:
                pend.append(pltpu.async_copy(
                    ef_hbm.at[pl.ds(base + (q + 2) * QK, QK)],
                    bufs[q % 2], sems[q % 2]))
            reg_scatter(idx2, td_v, 2 * q, 2 * q + 2)

    def pair(m, carry):
        k0 = 2 * m
        k1 = 2 * m + 1
        ck0 = slot_ck(k0)
        ck1 = slot_ck(k1)
        has1 = k1 < tw
        a_i0, a_t0 = issue_hdr(ck0, idx2s[0], td_vs[0])

        @pl.when(has1)
        def _():
            issue_hdr(ck1, idx2s[1], td_vs[1])

        chunk_work(ck0, idx2s[0], td_vs[0], a_i0, a_t0)

        @pl.when(has1)
        def _():
            pltpu.make_async_copy(
                row2_hbm.at[pl.ds(ck1 * (CHK // QK), CHK // QK)],
                idx2s[1], sem_i).wait()
            pltpu.make_async_copy(
                tdp_hbm.at[:, pl.ds(ck1 * CHK, CHK)], td_vs[1],
                sem_t).wait()
            base = ck1 * CHK
            bufs = (data0, data1)
            sems = (sem_d0, sem_d1)
            pend = [
                pltpu.async_copy(ef_hbm.at[pl.ds(base, QK)], data0, sem_d0),
                pltpu.async_copy(ef_hbm.at[pl.ds(base + QK, QK)], data1,
                                 sem_d1),
            ]
            nq = CHK // QK
            for q in range(nq):
                pend[q].wait()
                pltpu.sync_copy(bufs[q % 2], acch_sh.at[idx2s[1].at[q]],
                                add=True)
                if q + 2 < nq:
                    pend.append(pltpu.async_copy(
                        ef_hbm.at[pl.ds(base + (q + 2) * QK, QK)],
                        bufs[q % 2], sems[q % 2]))
                reg_scatter(idx2s[1], td_vs[1], 2 * q, 2 * q + 2)

        return carry

    lax.fori_loop(0, (NFULL + 2) // 2, pair, 0)

    pltpu.sync_copy(acc4_v, out4.at[wid])
    plsc.subcore_barrier()

    @pl.when(sid < NS - 1)
    def _():
        off = sid * RPT
        pltpu.sync_copy(acch_sh.at[pl.ds(off, RPT)],
                        outh.at[cid, pl.ds(off, RPT)])

    @pl.when(sid == NS - 1)
    def _():
        off = (NS - 1) * RPT
        pltpu.sync_copy(acch_sh.at[pl.ds(off, RPT15)],
                        outh.at[cid, pl.ds(off, RPT15)])


def _silu(x):
    return x * (1.0 / (1.0 + jnp.exp(-x)))


def _bdot(a, b):
    return jnp.dot(a.astype(jnp.bfloat16), b,
                   preferred_element_type=jnp.float32)


T_E = 3200
G_E = ES // T_E


def _edge_body(hr, hc, drp, eap, w1h, w1c, w4, w1e, be1, we2, be2, wc1, bc1,
               wc2, ef_out, tdp_out):
    drb = drp[...]
    x = (_bdot(hr[...], w1h[...])
         + _bdot(hc[...], w1c[...])
         + lax.dot_general(drb, w4[...], (((0,), (0,)), ((), ())),
                           preferred_element_type=jnp.float32)
         + lax.dot_general(eap[...], w1e[...], (((0,), (0,)), ((), ())),
                           preferred_element_type=jnp.float32)
         + be1[...])
    x = _silu(x)
    ef = _silu(_bdot(x, we2[...]) + be2[...])
    cf = _silu(_bdot(ef, wc1[...]) + bc1[...])
    s = lax.dot_general(wc2[...], cf, (((1,), (1,)), ((), ())),
                        preferred_element_type=jnp.float32)
    tr = jnp.clip(drb[:3, :] * s, -100.0, 100.0)
    ef_out[...] = ef
    tdp_out[:3, :] = tr
    tdp_out[3:4, :] = jnp.ones((1, T_E), jnp.float32)


def _edge_mlp(hr, hc, drp, eap, w1h, w1c, w4, w1e, be1, we2, be2, wc1, bc1,
              wc2):
    full = lambda shape: pl.BlockSpec(shape, lambda i: (0, 0))
    return pl.pallas_call(
        _edge_body,
        grid=(G_E,),
        in_specs=[
            pl.BlockSpec((T_E, D), lambda i: (i, 0)),
            pl.BlockSpec((T_E, D), lambda i: (i, 0)),
            pl.BlockSpec((4, T_E), lambda i: (0, i)),
            pl.BlockSpec((D_EDGE, T_E), lambda i: (0, i)),
            full((D, HID)),
            full((D, HID)),
            full((4, HID)),
            full((D_EDGE, HID)),
            full((1, HID)),
            full((HID, HID)),
            full((1, HID)),
            full((HID, HID)),
            full((1, HID)),
            full((1, HID)),
        ],
        out_specs=[
            pl.BlockSpec((T_E, D), lambda i: (i, 0)),
            pl.BlockSpec((4, T_E), lambda i: (0, i)),
        ],
        out_shape=[
            jax.ShapeDtypeStruct((ES, D), jnp.float32),
            jax.ShapeDtypeStruct((4, ES), jnp.float32),
        ],
        compiler_params=pltpu.CompilerParams(
            dimension_semantics=("arbitrary",)),
    )(hr, hc, drp, eap, w1h, w1c, w4, w1e, be1, we2, be2, wc1, bc1, wc2)


T_N = 2000
G_N = N // T_N


def _node_body(h, coordp, *rest):
    acchs = rest[:NSEG]
    acc4s = rest[NSEG:2 * NSEG]
    wn1h, wn1a, bn1, wn2, bn2, h_out, coordp_out = rest[2 * NSEG:]
    aggh = acchs[0][0] + acchs[0][1]
    for s in range(1, NSEG):
        aggh = aggh + acchs[s][0] + acchs[s][1]
    y = _silu(_bdot(h[...], wn1h[...]) + _bdot(aggh, wn1a[...]) + bn1[...])
    h_out[...] = h[...] + _bdot(y, wn2[...]) + bn2[...]

    @pl.when(pl.program_id(0) == 0)
    def _():
        a4 = acc4s[0][0]
        for k in range(1, NW):
            a4 = a4 + acc4s[0][k]
        for s in range(1, NSEG):
            for k in range(NW):
                a4 = a4 + acc4s[s][k]
        num = a4[:3, :]
        cnt = a4[3:4, :]
        coordp_out[...] = coordp[...] + num / jnp.maximum(cnt, 1.0)


def _node_mlp(h, coordp, acchs, acc4s, wn1h, wn1a, bn1, wn2, bn2):
    full = lambda shape: pl.BlockSpec(shape, lambda i: (0, 0))
    acch_spec = pl.BlockSpec((NC, T_N, D), lambda i: (0, i, 0))
    acc4_spec = pl.BlockSpec((NW, 4, N), lambda i: (0, 0, 0))
    return pl.pallas_call(
        _node_body,
        grid=(G_N,),
        in_specs=[
            pl.BlockSpec((T_N, D), lambda i: (i, 0)),
            pl.BlockSpec((3, N), lambda i: (0, 0)),
        ] + [acch_spec] * NSEG + [acc4_spec] * NSEG + [
            full((D, HID)),
            full((D, HID)),
            full((1, HID)),
            full((HID, D)),
            full((1, D)),
        ],
        out_specs=[
            pl.BlockSpec((T_N, D), lambda i: (i, 0)),
            pl.BlockSpec((3, N), lambda i: (0, 0)),
        ],
        out_shape=[
            jax.ShapeDtypeStruct((N, D), jnp.float32),
            jax.ShapeDtypeStruct((3, N), jnp.float32),
        ],
        compiler_params=pltpu.CompilerParams(
            dimension_semantics=("arbitrary",)),
    )(h, coordp, *acchs, *acc4s, wn1h, wn1a, bn1, wn2, bn2)


def kernel(h, coord, edge_attr, W_e1, b_e1, W_e2, b_e2, W_n1, b_n1, W_n2,
           b_n2, W_c1, b_c1, W_c2, edge_index):
    bf = jnp.bfloat16
    coordf = coord.reshape(-1)
    eap = edge_attr.T

    w4 = jnp.concatenate(
        [jnp.zeros((3, HID), jnp.float32), W_e1[2 * D:2 * D + 1]], axis=0)
    ew = (W_e1[:D].astype(bf), W_e1[D:2 * D].astype(bf), w4,
          W_e1[2 * D + 1:], b_e1.reshape(1, HID),
          W_e2.astype(bf), b_e2.reshape(1, HID),
          W_c1.astype(bf), b_c1.reshape(1, HID), W_c2.reshape(1, HID))

    z2 = jnp.zeros((RPT, D), jnp.float32)
    zf = jnp.zeros((4 * N,), jnp.float32)

    rows = [edge_index[0, s * ES:(s + 1) * ES] for s in range(NSEG)]
    cols = [edge_index[1, s * ES:(s + 1) * ES] for s in range(NSEG)]
    eaps = [eap[:, s * ES:(s + 1) * ES] for s in range(NSEG)]

    gat = [_gather(h, coordf, rows[s], cols[s]) for s in range(NSEG)]
    edg = [_edge_mlp(gat[s][0], gat[s][1], gat[s][2], eaps[s], *ew)
           for s in range(NSEG)]
    sca = [_scatter(edg[s][0], edg[s][1], rows[s].reshape(ES // QK, QK),
                    z2, zf) for s in range(NSEG)]

    h_out, coordp_out = _node_mlp(
        h, coord.T, [s[0] for s in sca],
        [s[1].reshape(NW, 4, N) for s in sca],
        W_n1[:D].astype(bf), W_n1[D:].astype(bf), b_n1.reshape(1, HID),
        W_n2.astype(bf), b_n2.reshape(1, D))
    return (h_out, coordp_out.T)
